# baseline (speedup 1.0000x reference)
import functools

import jax
import jax.numpy as jnp
from jax import lax
from jax.experimental import pallas as pl
from jax.experimental.pallas import tpu as pltpu
from jax.experimental.pallas import tpu_sc as plsc

RANK = 32
BATCH = 16384
PACK = 128 // RANK

_INFO = plsc.get_sparse_core_info()
_NC = _INFO.num_cores
_NS = _INFO.num_subcores
_NW = _NC * _NS
_BPW = BATCH // _NW
_CH = 128
_NCHUNK = _BPW // _CH


_PCOLS = 4096
_PROWS = _PCOLS // PACK


def _pack_body(inT_ref, o_ref):
    x = inT_ref[...]
    xt = jnp.swapaxes(x, 0, 1)
    o_ref[...] = jnp.concatenate(
        [xt[j * _PROWS:(j + 1) * _PROWS, :] for j in range(PACK)], axis=1)


def _pack(tableT):
    rank, n = tableT.shape
    grid = -(-n // _PCOLS)
    return pl.pallas_call(
        _pack_body,
        grid=(grid,),
        in_specs=[pl.BlockSpec((rank, _PCOLS), lambda i: (0, i))],
        out_specs=pl.BlockSpec((_PROWS, 128), lambda i: (i, 0)),
        out_shape=jax.ShapeDtypeStruct((grid * _PROWS, 128), jnp.float32),
    )(tableT)


def _sc_gather_body(tidx_hbm, ridx_hbm, cidx_hbm, tt_hbm, ut_hbm, it_hbm,
                    out_t, out_u, out_i,
                    ti_v, ri_v, ci_v, g_v, buf0, buf1, sem0, sem1):
    wid = lax.axis_index("s") * _NC + lax.axis_index("c")
    base = wid * _BPW
    for j in range(_NCHUNK):
        sl = pl.ds(base + j * _CH, _CH)
        pltpu.sync_copy(tidx_hbm.at[sl], ti_v.at[j])
        pltpu.sync_copy(ridx_hbm.at[sl], ri_v.at[j])
        pltpu.sync_copy(cidx_hbm.at[sl], ci_v.at[j])
    for tab, idx_v, out in ((tt_hbm, ti_v, out_t),
                            (ut_hbm, ri_v, out_u),
                            (it_hbm, ci_v, out_i)):
        for j in range(_NCHUNK):
            gj = g_v.at[j]
            for k in range(_CH // 16):
                r = idx_v[j, pl.ds(k * 16, 16)]
                gj[pl.ds(k * 16, 16)] = ((r >> 12) << 10) + (r & 1023)
        bufs = (buf0, buf1)
        sems = (sem0, sem1)
        copies = [None, None]
        for j in range(_NCHUNK):
            b = j % 2
            if copies[b] is not None:
                copies[b].wait()
                pltpu.sync_copy(bufs[b], out.at[pl.ds(base + (j - 2) * _CH, _CH)])
            copies[b] = pltpu.async_copy(tab.at[g_v.at[j]], bufs[b], sems[b])
        for j in range(_NCHUNK - 2, _NCHUNK):
            b = j % 2
            copies[b].wait()
            pltpu.sync_copy(bufs[b], out.at[pl.ds(base + j * _CH, _CH)])


@jax.jit
def _sc_gather(tIdx, rIdx, cIdx, tt_p, ut_p, it_p):
    mesh = plsc.VectorSubcoreMesh(core_axis_name="c", subcore_axis_name="s")
    f = functools.partial(
        pl.kernel,
        mesh=mesh,
        out_type=(
            jax.ShapeDtypeStruct((BATCH, 128), jnp.float32),
            jax.ShapeDtypeStruct((BATCH, 128), jnp.float32),
            jax.ShapeDtypeStruct((BATCH, 128), jnp.float32),
        ),
        scratch_types=[
            pltpu.VMEM((_NCHUNK, _CH), jnp.int32),
            pltpu.VMEM((_NCHUNK, _CH), jnp.int32),
            pltpu.VMEM((_NCHUNK, _CH), jnp.int32),
            pltpu.VMEM((_NCHUNK, _CH), jnp.int32),
            pltpu.VMEM((_CH, 128), jnp.float32),
            pltpu.VMEM((_CH, 128), jnp.float32),
            pltpu.SemaphoreType.DMA,
            pltpu.SemaphoreType.DMA,
        ],
    )(_sc_gather_body)
    return f(tIdx, rIdx, cIdx, tt_p, ut_p, it_p)


def _tc_body(gt_ref, gu_ref, gi_ref, st_ref, su_ref, si_ref,
             wt_ref, wu_ref, wi_ref, bt_ref, bu_ref, bi_ref, o_ref):
    def select(g_ref, s_ref):
        g = g_ref[...]
        s = s_ref[...]
        acc = jnp.zeros((g.shape[0], RANK), jnp.float32)
        for j in range(PACK):
            seg = g[:, j * RANK:(j + 1) * RANK]
            acc = acc + jnp.where(s == j, seg, 0.0)
        return acc

    t = jnp.dot(select(gt_ref, st_ref), wt_ref[...],
                preferred_element_type=jnp.float32) + bt_ref[...]
    u = jnp.dot(select(gu_ref, su_ref), wu_ref[...],
                preferred_element_type=jnp.float32) + bu_ref[...]
    i = jnp.dot(select(gi_ref, si_ref), wi_ref[...],
                preferred_element_type=jnp.float32) + bi_ref[...]
    o_ref[...] = jnp.sum(t * u * i, axis=-1)


_TC_BLOCK = 2048


@jax.jit
def _tc_combine(gt, gu, gi, st, su, si, WtT, WuT, WiT, bt, bu, bi):
    grid = BATCH // _TC_BLOCK
    g_spec = pl.BlockSpec((_TC_BLOCK, 128), lambda i: (i, 0))
    s_spec = pl.BlockSpec((_TC_BLOCK, 1), lambda i: (i, 0))
    w_spec = pl.BlockSpec((RANK, RANK), lambda i: (0, 0))
    b_spec = pl.BlockSpec((1, RANK), lambda i: (0, 0))
    return pl.pallas_call(
        _tc_body,
        grid=(grid,),
        in_specs=[g_spec, g_spec, g_spec, s_spec, s_spec, s_spec,
                  w_spec, w_spec, w_spec, b_spec, b_spec, b_spec],
        out_specs=pl.BlockSpec((_TC_BLOCK,), lambda i: (i,)),
        out_shape=jax.ShapeDtypeStruct((BATCH,), jnp.float32),
    )(gt, gu, gi, st, su, si, WtT, WuT, WiT, bt, bu, bi)


def kernel(tIdx, rIdx, cIdx, time_table, user_table, item_table,
           Wt, bt, Wu, bu, Wi, bi):
    tt_p = _pack(time_table.T)
    ut_p = _pack(user_table.T)
    it_p = _pack(item_table.T)
    gt, gu, gi = _sc_gather(tIdx, rIdx, cIdx, tt_p, ut_p, it_p)
    st = ((tIdx >> 10) & 3).reshape(BATCH, 1)
    su = ((rIdx >> 10) & 3).reshape(BATCH, 1)
    si = ((cIdx >> 10) & 3).reshape(BATCH, 1)
    return _tc_combine(gt, gu, gi, st, su, si, Wt.T, Wu.T, Wi.T,
                       bt.reshape(1, RANK), bu.reshape(1, RANK),
                       bi.reshape(1, RANK))


# speedup vs baseline: 1.1344x; 1.1344x over previous
import functools

import jax
import jax.numpy as jnp
from jax import lax
from jax.experimental import pallas as pl
from jax.experimental.pallas import tpu as pltpu
from jax.experimental.pallas import tpu_sc as plsc

RANK = 32
BATCH = 16384

_INFO = plsc.get_sparse_core_info()
_NC = _INFO.num_cores
_NS = _INFO.num_subcores
_NW = _NC * _NS
_BPW = BATCH // _NW
_CH = 64


def _gather_rows(tab_hbm, idx_v, buf_v, out, base, sem):
    def chunk(c, carry):
        for k in range(_CH // 16):
            v = idx_v[pl.ds(c * _CH + k * 16, 16)]
            for l in range(16):
                pltpu.async_copy(tab_hbm.at[pl.ds(v[l], 1)],
                                 buf_v.at[pl.ds(k * 16 + l, 1)], sem)
        pltpu.make_async_copy(tab_hbm.at[pl.ds(0, _CH)], buf_v, sem).wait()
        pltpu.sync_copy(buf_v, out.at[pl.ds(base + c * _CH, _CH)])
        return carry

    lax.fori_loop(0, _BPW // _CH, chunk, 0)


def _sc_gather_ti_body(tidx_hbm, cidx_hbm, tt_hbm, it_hbm, out_t, out_i,
                       ti_v, ci_v, tr_v, ir_v, sem):
    wid = lax.axis_index("s") * _NC + lax.axis_index("c")
    base = wid * _BPW
    sl = pl.ds(base, _BPW)
    pltpu.sync_copy(tidx_hbm.at[sl], ti_v)
    pltpu.sync_copy(cidx_hbm.at[sl], ci_v)
    _gather_rows(tt_hbm, ti_v, tr_v, out_t, base, sem)
    _gather_rows(it_hbm, ci_v, ir_v, out_i, base, sem)


def _sc_gather_u_body(ridx_hbm, ut_hbm, out_u, ri_v, ur_v, sem):
    wid = lax.axis_index("s") * _NC + lax.axis_index("c")
    base = wid * _BPW
    pltpu.sync_copy(ridx_hbm.at[pl.ds(base, _BPW)], ri_v)
    _gather_rows(ut_hbm, ri_v, ur_v, out_u, base, sem)


_EMB = jax.ShapeDtypeStruct((BATCH, RANK), jnp.float32)
_MESH = dict(core_axis_name="c", subcore_axis_name="s")


@jax.jit
def _sc_gather_ti(tIdx, cIdx, time_table, item_table):
    f = functools.partial(
        pl.kernel,
        mesh=plsc.VectorSubcoreMesh(**_MESH),
        out_type=(_EMB, _EMB),
        scratch_types=[
            pltpu.VMEM((_BPW,), jnp.int32),
            pltpu.VMEM((_BPW,), jnp.int32),
            pltpu.VMEM((_CH, RANK), jnp.float32),
            pltpu.VMEM((_CH, RANK), jnp.float32),
            pltpu.SemaphoreType.DMA,
        ],
    )(_sc_gather_ti_body)
    return f(tIdx, cIdx, time_table, item_table)


@jax.jit
def _sc_gather_u(rIdx, user_table):
    f = functools.partial(
        pl.kernel,
        mesh=plsc.VectorSubcoreMesh(**_MESH),
        out_type=_EMB,
        scratch_types=[
            pltpu.VMEM((_BPW,), jnp.int32),
            pltpu.VMEM((_CH, RANK), jnp.float32),
            pltpu.SemaphoreType.DMA,
        ],
    )(_sc_gather_u_body)
    return f(rIdx, user_table)


def _tc_body(et_ref, eu_ref, ei_ref, wt_ref, wu_ref, wi_ref,
             bt_ref, bu_ref, bi_ref, o_ref):
    t = jnp.dot(et_ref[...], wt_ref[...], preferred_element_type=jnp.float32) + bt_ref[...]
    u = jnp.dot(eu_ref[...], wu_ref[...], preferred_element_type=jnp.float32) + bu_ref[...]
    i = jnp.dot(ei_ref[...], wi_ref[...], preferred_element_type=jnp.float32) + bi_ref[...]
    o_ref[...] = jnp.sum(t * u * i, axis=-1)


_TC_BLOCK = 2048


@jax.jit
def _tc_combine(et, eu, ei, WtT, WuT, WiT, bt, bu, bi):
    grid = BATCH // _TC_BLOCK
    emb_spec = pl.BlockSpec((_TC_BLOCK, RANK), lambda i: (i, 0))
    w_spec = pl.BlockSpec((RANK, RANK), lambda i: (0, 0))
    b_spec = pl.BlockSpec((1, RANK), lambda i: (0, 0))
    return pl.pallas_call(
        _tc_body,
        grid=(grid,),
        in_specs=[emb_spec, emb_spec, emb_spec, w_spec, w_spec, w_spec,
                  b_spec, b_spec, b_spec],
        out_specs=pl.BlockSpec((_TC_BLOCK,), lambda i: (i,)),
        out_shape=jax.ShapeDtypeStruct((BATCH,), jnp.float32),
    )(et, eu, ei, WtT, WuT, WiT, bt, bu, bi)


def kernel(tIdx, rIdx, cIdx, time_table, user_table, item_table,
           Wt, bt, Wu, bu, Wi, bi):
    et, ei = _sc_gather_ti(tIdx, cIdx, time_table, item_table)
    eu = _sc_gather_u(rIdx, user_table)
    return _tc_combine(et, eu, ei, Wt.T, Wu.T, Wi.T,
                       bt.reshape(1, RANK), bu.reshape(1, RANK),
                       bi.reshape(1, RANK))


# speedup vs baseline: 1.1450x; 1.0093x over previous
import functools

import jax
import jax.numpy as jnp
from jax import lax
from jax.experimental import pallas as pl
from jax.experimental.pallas import tpu as pltpu
from jax.experimental.pallas import tpu_sc as plsc

RANK = 32
BATCH = 16384

_INFO = plsc.get_sparse_core_info()
_NC = _INFO.num_cores
_NS = _INFO.num_subcores
_NW = _NC * _NS
_BPW = BATCH // _NW
_CH = 64


def _gather_rows(tab_hbm, idx_v, buf_v, out, base, sem):
    def chunk(c, carry):
        for k in range(_CH // 16):
            v = idx_v[pl.ds(c * _CH + k * 16, 16)]
            for l in range(16):
                pltpu.async_copy(tab_hbm.at[pl.ds(v[l], 1)],
                                 buf_v.at[pl.ds(k * 16 + l, 1)], sem)
        pltpu.make_async_copy(tab_hbm.at[pl.ds(0, _CH)], buf_v, sem).wait()
        pltpu.sync_copy(buf_v, out.at[pl.ds(base + c * _CH, _CH)])
        return carry

    lax.fori_loop(0, _BPW // _CH, chunk, 0)


def _sc_gather_ti_body(tidx_hbm, cidx_hbm, tt_hbm, it_hbm, out_t, out_i,
                       ti_v, ci_v, tr_v, ir_v, sem):
    wid = lax.axis_index("s") * _NC + lax.axis_index("c")
    base = wid * _BPW
    sl = pl.ds(base, _BPW)
    pltpu.sync_copy(tidx_hbm.at[sl], ti_v)
    pltpu.sync_copy(cidx_hbm.at[sl], ci_v)
    _gather_rows(tt_hbm, ti_v, tr_v, out_t, base, sem)
    _gather_rows(it_hbm, ci_v, ir_v, out_i, base, sem)


def _sc_gather_u_body(ridx_hbm, ut_hbm, out_u, ri_v, *rest):
    bufs, sems = rest[:_BPW // _CH], rest[_BPW // _CH:]
    wid = lax.axis_index("s") * _NC + lax.axis_index("c")
    base = wid * _BPW
    pltpu.sync_copy(ridx_hbm.at[pl.ds(base, _BPW)], ri_v)
    for c in range(_BPW // _CH):
        for k in range(_CH // 16):
            v = ri_v[pl.ds(c * _CH + k * 16, 16)]
            for l in range(16):
                pltpu.async_copy(ut_hbm.at[pl.ds(v[l], 1)],
                                 bufs[c].at[pl.ds(k * 16 + l, 1)], sems[c])
    for c in range(_BPW // _CH):
        pltpu.make_async_copy(ut_hbm.at[pl.ds(0, _CH)], bufs[c], sems[c]).wait()
        pltpu.sync_copy(bufs[c], out_u.at[pl.ds(base + c * _CH, _CH)])


_EMB = jax.ShapeDtypeStruct((BATCH, RANK), jnp.float32)
_MESH = dict(core_axis_name="c", subcore_axis_name="s")


@jax.jit
def _sc_gather_ti(tIdx, cIdx, time_table, item_table):
    f = functools.partial(
        pl.kernel,
        mesh=plsc.VectorSubcoreMesh(**_MESH),
        out_type=(_EMB, _EMB),
        scratch_types=[
            pltpu.VMEM((_BPW,), jnp.int32),
            pltpu.VMEM((_BPW,), jnp.int32),
            pltpu.VMEM((_CH, RANK), jnp.float32),
            pltpu.VMEM((_CH, RANK), jnp.float32),
            pltpu.SemaphoreType.DMA,
        ],
    )(_sc_gather_ti_body)
    return f(tIdx, cIdx, time_table, item_table)


@jax.jit
def _sc_gather_u(rIdx, user_table):
    f = functools.partial(
        pl.kernel,
        mesh=plsc.VectorSubcoreMesh(**_MESH),
        out_type=_EMB,
        scratch_types=(
            [pltpu.VMEM((_BPW,), jnp.int32)]
            + [pltpu.VMEM((_CH, RANK), jnp.float32)] * (_BPW // _CH)
            + [pltpu.SemaphoreType.DMA] * (_BPW // _CH)
        ),
    )(_sc_gather_u_body)
    return f(rIdx, user_table)


def _tc_body(et_ref, eu_ref, ei_ref, wt_ref, wu_ref, wi_ref,
             bt_ref, bu_ref, bi_ref, o_ref):
    t = jnp.dot(et_ref[...], wt_ref[...], preferred_element_type=jnp.float32) + bt_ref[...]
    u = jnp.dot(eu_ref[...], wu_ref[...], preferred_element_type=jnp.float32) + bu_ref[...]
    i = jnp.dot(ei_ref[...], wi_ref[...], preferred_element_type=jnp.float32) + bi_ref[...]
    o_ref[...] = jnp.sum(t * u * i, axis=-1)


_TC_BLOCK = 2048


@jax.jit
def _tc_combine(et, eu, ei, WtT, WuT, WiT, bt, bu, bi):
    grid = BATCH // _TC_BLOCK
    emb_spec = pl.BlockSpec((_TC_BLOCK, RANK), lambda i: (i, 0))
    w_spec = pl.BlockSpec((RANK, RANK), lambda i: (0, 0))
    b_spec = pl.BlockSpec((1, RANK), lambda i: (0, 0))
    return pl.pallas_call(
        _tc_body,
        grid=(grid,),
        in_specs=[emb_spec, emb_spec, emb_spec, w_spec, w_spec, w_spec,
                  b_spec, b_spec, b_spec],
        out_specs=pl.BlockSpec((_TC_BLOCK,), lambda i: (i,)),
        out_shape=jax.ShapeDtypeStruct((BATCH,), jnp.float32),
    )(et, eu, ei, WtT, WuT, WiT, bt, bu, bi)


def kernel(tIdx, rIdx, cIdx, time_table, user_table, item_table,
           Wt, bt, Wu, bu, Wi, bi):
    et, ei = _sc_gather_ti(tIdx, cIdx, time_table, item_table)
    eu = _sc_gather_u(rIdx, user_table)
    return _tc_combine(et, eu, ei, Wt.T, Wu.T, Wi.T,
                       bt.reshape(1, RANK), bu.reshape(1, RANK),
                       bi.reshape(1, RANK))
